# Initial kernel scaffold; baseline (speedup 1.0000x reference)
import jax
import jax.numpy as jnp
from jax.experimental import pallas as pl


def kernel(src_hashes, pos_dst_hashes, neg_dst_hashes, W_src, W_tgt):
    raise NotImplementedError("write your pallas kernel here")


# speedup vs baseline: 2.2609x; 2.2609x over previous
import functools

import jax
import jax.numpy as jnp
from jax import lax
from jax.experimental import pallas as pl
from jax.experimental.pallas import tpu as pltpu
from jax.experimental.pallas import tpu_sc as plsc

B = 16384
M = 16
NEG = 8
D = 32
TOTAL_BUCKETS = 16 * (2 ** 16)

NC = 2
NS = 16
NW = NC * NS
EPW = B // NW

CB = 8
CHUNKS = EPW // CB
ROWS_SRC = CB * M
ROWS_NEG = CB * NEG * M


def _tree_sum(vals):
    while len(vals) > 1:
        vals = [vals[i] + vals[i + 1] for i in range(0, len(vals) - 1, 2)] + (
            [vals[-1]] if len(vals) % 2 else [])
    return vals[0]


def _sc_body(src_idx, pos_idx, neg_idx, w_src, w_tgt,
             su_out, tp_out, tn_out,
             idx_sp, idx_ng, rows_src, rows_pos, rows_neg,
             su_st, tp_st, tn_st, sem, osem):
    wid = lax.axis_index("s") * NC + lax.axis_index("c")

    def chunk(g, _):
        sp_row = wid * (EPW * M // 128) + g
        ng_row = wid * (EPW * NEG * M // 128) + g * (ROWS_NEG // 128)
        pltpu.sync_copy(src_idx.at[pl.ds(sp_row, 1)], idx_sp.at[pl.ds(0, 1)])
        pltpu.sync_copy(pos_idx.at[pl.ds(sp_row, 1)], idx_sp.at[pl.ds(1, 1)])
        pltpu.sync_copy(neg_idx.at[pl.ds(ng_row, ROWS_NEG // 128)], idx_ng)

        cps = [
            pltpu.async_copy(w_src.at[idx_sp.at[0]], rows_src, sem),
            pltpu.async_copy(w_tgt.at[idx_sp.at[1]], rows_pos, sem),
        ]
        for j in range(ROWS_NEG // 128):
            cps.append(pltpu.async_copy(
                w_tgt.at[idx_ng.at[j]],
                rows_neg.at[pl.ds(j * 128, 128)], sem))
        for c in cps:
            c.wait()

        for e in range(CB):
            for h in range(2):
                cs = pl.ds(h * 16, 16)
                su_st[e, cs] = _tree_sum(
                    [rows_src[e * M + m, cs] for m in range(M)])
                tp_st[e, cs] = _tree_sum(
                    [rows_pos[e * M + m, cs] for m in range(M)])
                for n in range(NEG):
                    r0 = (e * NEG + n) * M
                    tn_st[e * NEG + n, cs] = _tree_sum(
                        [rows_neg[r0 + m, cs] for m in range(M)])

        base = wid * EPW + g * CB
        ocs = [
            pltpu.async_copy(su_st, su_out.at[pl.ds(base, CB)], osem),
            pltpu.async_copy(tp_st, tp_out.at[pl.ds(base, CB)], osem),
            pltpu.async_copy(tn_st, tn_out.at[pl.ds(base * NEG, CB * NEG)],
                             osem),
        ]
        for c in ocs:
            c.wait()
        return ()

    lax.fori_loop(0, CHUNKS, chunk, (), unroll=False)


@functools.partial(jax.jit, static_argnums=())
def _sc_gather_sums(src_idx, pos_idx, neg_idx, w_src, w_tgt):
    f32 = jnp.float32
    return pl.kernel(
        _sc_body,
        out_type=[
            jax.ShapeDtypeStruct((B, D), f32),
            jax.ShapeDtypeStruct((B, D), f32),
            jax.ShapeDtypeStruct((B * NEG, D), f32),
        ],
        mesh=plsc.VectorSubcoreMesh(
            core_axis_name="c", subcore_axis_name="s",
            num_cores=NC, num_subcores=NS),
        scratch_types=[
            pltpu.VMEM((2, 128), jnp.int32),
            pltpu.VMEM((ROWS_NEG // 128, 128), jnp.int32),
            pltpu.VMEM((ROWS_SRC, D), f32),
            pltpu.VMEM((ROWS_SRC, D), f32),
            pltpu.VMEM((ROWS_NEG, D), f32),
            pltpu.VMEM((CB, D), f32),
            pltpu.VMEM((CB, D), f32),
            pltpu.VMEM((CB * NEG, D), f32),
            pltpu.SemaphoreType.DMA,
            pltpu.SemaphoreType.DMA,
        ],
        compiler_params=pltpu.CompilerParams(use_tc_tiling_on_sc=False),
    )(src_idx, pos_idx, neg_idx, w_src, w_tgt)


LOSS_CHUNK = 1024


def _loss_body(su_ref, tp_ref, tn_ref, out_ref):
    i = pl.program_id(0)
    su = su_ref[...]
    tp = tp_ref[...]
    tn = tn_ref[...].reshape(LOSS_CHUNK, NEG, D)
    scale = 1.0 / (M * M)
    pos = jnp.sum(su * tp, axis=1) * scale
    neg = jnp.sum(tn * su[:, None, :], axis=2) * scale
    losses = (-jax.nn.log_sigmoid(pos)
              - jnp.sum(jax.nn.log_sigmoid(-neg), axis=1))
    part = jnp.sum(losses) * (1.0 / B)

    @pl.when(i == 0)
    def _():
        out_ref[0, 0] = 0.0

    out_ref[0, 0] += part


def _tc_loss(su, tp, tn):
    out = pl.pallas_call(
        _loss_body,
        grid=(B // LOSS_CHUNK,),
        in_specs=[
            pl.BlockSpec((LOSS_CHUNK, D), lambda i: (i, 0)),
            pl.BlockSpec((LOSS_CHUNK, D), lambda i: (i, 0)),
            pl.BlockSpec((LOSS_CHUNK * NEG, D), lambda i: (i, 0)),
        ],
        out_specs=pl.BlockSpec(
            (1, 1), lambda i: (0, 0), memory_space=pltpu.SMEM),
        out_shape=jax.ShapeDtypeStruct((1, 1), jnp.float32),
    )(su, tp, tn)
    return out[0, 0]


def kernel(src_hashes, pos_dst_hashes, neg_dst_hashes, W_src, W_tgt):
    src_i = src_hashes.astype(jnp.int32).reshape(B * M // 128, 128)
    pos_i = pos_dst_hashes.astype(jnp.int32).reshape(B * M // 128, 128)
    neg_i = neg_dst_hashes.astype(jnp.int32).reshape(B * NEG * M // 128, 128)
    su, tp, tn = _sc_gather_sums(src_i, pos_i, neg_i, W_src, W_tgt)
    return _tc_loss(su, tp, tn)
